# baseline (speedup 1.0000x reference)
import jax
import jax.numpy as jnp
from jax import lax
from jax.experimental import pallas as pl
from jax.experimental.pallas import tpu as pltpu
from jax.experimental.pallas import tpu_sc as plsc

N = 10000
E = 320000
B = 64
DX = 128
DE = 16
DU = 128
H1 = 512
H2 = 256

NC = 2
NS = 16
E_PER_W = E // (NC * NS)
KG = 2000
S_PER_W = E_PER_W // KG


def _sc_edge_kernel(row_hbm, attr_hbm, batch_hbm, out_sum, out_cnt,
                    batch_v, rows_v, attr_v, acc_v, cnt_v, tmp_v, tmpc_v,
                    sum_sh, cnt_sh):
    c = lax.axis_index("c")
    s = lax.axis_index("s")
    wid = s * NC + c

    def zrow(i, _):
        acc_v[i, :] = jnp.zeros((16,), jnp.float32)
        return 0
    lax.fori_loop(0, B, zrow, 0)
    for i in range(B // 16):
        cnt_v[pl.ds(16 * i, 16)] = jnp.zeros((16,), jnp.float32)

    pltpu.sync_copy(batch_hbm, batch_v)
    ones16 = jnp.ones((16,), jnp.float32)
    iota16 = lax.broadcasted_iota(jnp.int32, (16,), 0)
    base = wid * E_PER_W

    def chunk(si, _):
        off = base + si * KG
        pltpu.sync_copy(row_hbm.at[pl.ds(off, KG)], rows_v)
        pltpu.sync_copy(attr_hbm.at[pl.ds(off, KG)], attr_v)

        def grp(i, _):
            r = rows_v[pl.ds(16 * i, 16)]
            seg = plsc.load_gather(batch_v, [r])
            plsc.addupdate_scatter(cnt_v, [seg], ones16)
            ridx = jnp.full((16,), 16, jnp.int32) * i + iota16
            for f in range(DE):
                cidx = jnp.full((16,), f, jnp.int32)
                col = plsc.load_gather(attr_v, [ridx, cidx])
                plsc.addupdate_scatter(acc_v, [seg, cidx], col)
            return 0
        lax.fori_loop(0, KG // 16, grp, 0)
        return 0
    lax.fori_loop(0, S_PER_W, chunk, 0)

    pltpu.sync_copy(acc_v, sum_sh.at[s])
    pltpu.sync_copy(cnt_v, cnt_sh.at[s])
    plsc.subcore_barrier()

    @pl.when(s == 0)
    def _():
        def red(t, _):
            pltpu.sync_copy(sum_sh.at[t], tmp_v)
            def addrow(i, _):
                acc_v[i, :] = acc_v[i, :] + tmp_v[i, :]
                return 0
            lax.fori_loop(0, B, addrow, 0)
            pltpu.sync_copy(cnt_sh.at[t], tmpc_v)
            for i in range(B // 16):
                sl = pl.ds(16 * i, 16)
                cnt_v[sl] = cnt_v[sl] + tmpc_v[sl]
            return 0
        lax.fori_loop(1, NS, red, 0)
        pltpu.sync_copy(acc_v, out_sum.at[c])
        pltpu.sync_copy(cnt_v, out_cnt.at[c])


def _sc_edge(row, edge_attr, batch32):
    mesh = plsc.VectorSubcoreMesh(core_axis_name="c", subcore_axis_name="s")
    f32 = jnp.float32
    return pl.kernel(
        _sc_edge_kernel,
        out_type=(jax.ShapeDtypeStruct((NC, B, DE), f32),
                  jax.ShapeDtypeStruct((NC, B), f32)),
        mesh=mesh,
        compiler_params=pltpu.CompilerParams(
            needs_layout_passes=False, use_tc_tiling_on_sc=False),
        scratch_types=[
            pltpu.VMEM((N,), jnp.int32),
            pltpu.VMEM((KG,), jnp.int32),
            pltpu.VMEM((KG, DE), f32),
            pltpu.VMEM((B, DE), f32),
            pltpu.VMEM((B,), f32),
            pltpu.VMEM((B, DE), f32),
            pltpu.VMEM((B,), f32),
            pltpu.VMEM_SHARED((NS, B, DE), f32),
            pltpu.VMEM_SHARED((NS, B), f32),
        ],
    )(row, edge_attr, batch32)


NBN = N // 128
NTAIL = N - NBN * 128
NPAD = (NBN + 2) * 128


def _tc_fuse_kernel(x_ref, b_ref, u_ref, w1u_ref, w1x_ref, w1e_ref, b1_ref,
                    w2_ref, b2_ref, es_ref, ec_ref, out_ref):
    f32 = jnp.float32
    bins = lax.broadcasted_iota(jnp.int32, (B, 128), 0)
    ones_n = jnp.ones((128, DE), f32)

    def nstep(j, carry):
        nsum, ncnt = carry
        brow = b_ref[pl.ds(j, 1), :]
        oh = (jnp.broadcast_to(brow, (B, 128)) == bins).astype(f32)
        x_blk = x_ref[pl.ds(j * 128, 128), :]
        nsum = nsum + lax.dot_general(
            oh, x_blk, (((1,), (0,)), ((), ())), preferred_element_type=f32)
        ncnt = ncnt + lax.dot_general(
            oh, ones_n, (((1,), (0,)), ((), ())), preferred_element_type=f32)
        return nsum, ncnt

    nsum, ncnt = lax.fori_loop(
        0, NBN, nstep, (jnp.zeros((B, DX), f32), jnp.zeros((B, DE), f32)))

    btail = b_ref[pl.ds(NBN, 1), pl.ds(0, NTAIL)]
    oh_t = (jnp.broadcast_to(btail, (B, NTAIL))
            == lax.broadcasted_iota(jnp.int32, (B, NTAIL), 0)).astype(f32)
    x_t = x_ref[pl.ds(NBN * 128, NTAIL), :]
    nsum = nsum + lax.dot_general(
        oh_t, x_t, (((1,), (0,)), ((), ())), preferred_element_type=f32)
    ncnt = ncnt + lax.dot_general(
        oh_t, jnp.ones((NTAIL, DE), f32), (((1,), (0,)), ((), ())),
        preferred_element_type=f32)

    nmean = nsum / jnp.maximum(ncnt[:, 0:1], 1.0)

    es = es_ref[0] + es_ref[1]
    ec = ec_ref[0] + ec_ref[1]
    emean = es / jnp.maximum(ec, 1.0)

    z = (jnp.dot(u_ref[...], w1u_ref[...], preferred_element_type=f32)
         + jnp.dot(nmean, w1x_ref[...], preferred_element_type=f32)
         + jnp.dot(emean, w1e_ref[...], preferred_element_type=f32)
         + b1_ref[...])
    h = jnp.maximum(z, 0.0)
    out_ref[...] = jnp.dot(h, w2_ref[...], preferred_element_type=f32) \
        + b2_ref[...]


def _tc_fuse(x, bp, u, w1u, w1x, w1e, b1, w2, b2, esum, ecnt):
    return pl.pallas_call(
        _tc_fuse_kernel,
        out_shape=jax.ShapeDtypeStruct((B, H2), jnp.float32),
    )(x, bp, u, w1u, w1x, w1e, b1, w2, b2, esum, ecnt)


@jax.jit
def kernel(x, edge_index, edge_attr, u, batch, W1, b1, W2, b2):
    row = edge_index[0].astype(jnp.int32)
    batch32 = batch.astype(jnp.int32)

    esum, ecnt = _sc_edge(row, edge_attr, batch32)
    ecnt = ecnt.reshape(NC, B, 1)

    bp = jnp.pad(batch32, (0, NPAD - N), constant_values=-1).reshape(-1, 128)
    w1u = W1[:DU]
    w1x = W1[DU:DU + DX]
    w1e = W1[DU + DX:]
    return _tc_fuse(x, bp, u, w1u, w1x, w1e, b1.reshape(1, H1), W2,
                    b2.reshape(1, H2), esum, ecnt)


# speedup vs baseline: 1.3611x; 1.3611x over previous
import jax
import jax.numpy as jnp
from jax import lax
from jax.experimental import pallas as pl
from jax.experimental.pallas import tpu as pltpu
from jax.experimental.pallas import tpu_sc as plsc

N = 10000
E = 320000
B = 64
DX = 128
DE = 16
DU = 128
H1 = 512
H2 = 256

NC = 2
NS = 16
E_PER_W = E // (NC * NS)
KG = 2000
S_PER_W = E_PER_W // KG


def _sc_edge_kernel(row_hbm, attr_hbm, batch_hbm, out_sum, out_cnt,
                    batch_v, rows_v, attr_v, acc_v, acc2_v, cnt_v, tmp_v,
                    tmpc_v, sum_sh, cnt_sh):
    c = lax.axis_index("c")
    s = lax.axis_index("s")
    wid = s * NC + c

    def zrow(i, _):
        acc_v[i, :] = jnp.zeros((16,), jnp.float32)
        acc2_v[i, :] = jnp.zeros((16,), jnp.float32)
        return 0
    lax.fori_loop(0, B, zrow, 0)
    for i in range(B // 16):
        cnt_v[pl.ds(16 * i, 16)] = jnp.zeros((16,), jnp.float32)

    pltpu.sync_copy(batch_hbm, batch_v)
    ones16 = jnp.ones((16,), jnp.float32)
    iota16 = lax.broadcasted_iota(jnp.int32, (16,), 0)
    base = wid * E_PER_W

    def chunk(si, _):
        off = base + si * KG
        pltpu.sync_copy(row_hbm.at[pl.ds(off, KG)], rows_v)
        pltpu.sync_copy(attr_hbm.at[pl.ds(off, KG)], attr_v)

        def grp(i, _):
            r = rows_v[pl.ds(16 * i, 16)]
            seg = plsc.load_gather(batch_v, [r])
            plsc.addupdate_scatter(cnt_v, [seg], ones16)
            for kk in range(0, 16, 2):
                plsc.addupdate(acc_v.at[seg[kk]], attr_v[16 * i + kk, :])
                plsc.addupdate(acc2_v.at[seg[kk + 1]],
                               attr_v[16 * i + kk + 1, :])
            return 0
        lax.fori_loop(0, KG // 16, grp, 0)
        return 0
    lax.fori_loop(0, S_PER_W, chunk, 0)

    def mrow(i, _):
        acc_v[i, :] = acc_v[i, :] + acc2_v[i, :]
        return 0
    lax.fori_loop(0, B, mrow, 0)

    pltpu.sync_copy(acc_v, sum_sh.at[s])
    pltpu.sync_copy(cnt_v, cnt_sh.at[s])
    plsc.subcore_barrier()

    @pl.when(s == 0)
    def _():
        def red(t, _):
            pltpu.sync_copy(sum_sh.at[t], tmp_v)
            def addrow(i, _):
                acc_v[i, :] = acc_v[i, :] + tmp_v[i, :]
                return 0
            lax.fori_loop(0, B, addrow, 0)
            pltpu.sync_copy(cnt_sh.at[t], tmpc_v)
            for i in range(B // 16):
                sl = pl.ds(16 * i, 16)
                cnt_v[sl] = cnt_v[sl] + tmpc_v[sl]
            return 0
        lax.fori_loop(1, NS, red, 0)
        pltpu.sync_copy(acc_v, out_sum.at[c])
        pltpu.sync_copy(cnt_v, out_cnt.at[c])


def _sc_edge(row, edge_attr, batch32):
    mesh = plsc.VectorSubcoreMesh(core_axis_name="c", subcore_axis_name="s")
    f32 = jnp.float32
    return pl.kernel(
        _sc_edge_kernel,
        out_type=(jax.ShapeDtypeStruct((NC, B, DE), f32),
                  jax.ShapeDtypeStruct((NC, B), f32)),
        mesh=mesh,
        compiler_params=pltpu.CompilerParams(
            needs_layout_passes=False, use_tc_tiling_on_sc=False),
        scratch_types=[
            pltpu.VMEM((N,), jnp.int32),
            pltpu.VMEM((KG,), jnp.int32),
            pltpu.VMEM((KG, DE), f32),
            pltpu.VMEM((B, DE), f32),
            pltpu.VMEM((B, DE), f32),
            pltpu.VMEM((B,), f32),
            pltpu.VMEM((B, DE), f32),
            pltpu.VMEM((B,), f32),
            pltpu.VMEM_SHARED((NS, B, DE), f32),
            pltpu.VMEM_SHARED((NS, B), f32),
        ],
    )(row, edge_attr, batch32)


NBN = N // 128
NTAIL = N - NBN * 128
NPAD = (NBN + 2) * 128


def _tc_fuse_kernel(x_ref, b_ref, u_ref, w1u_ref, w1x_ref, w1e_ref, b1_ref,
                    w2_ref, b2_ref, es_ref, ec_ref, out_ref):
    f32 = jnp.float32
    bins = lax.broadcasted_iota(jnp.int32, (B, 128), 0)
    ones_n = jnp.ones((128, DE), f32)

    def nstep(j, carry):
        nsum, ncnt = carry
        brow = b_ref[pl.ds(j, 1), :]
        oh = (jnp.broadcast_to(brow, (B, 128)) == bins).astype(f32)
        x_blk = x_ref[pl.ds(j * 128, 128), :]
        nsum = nsum + lax.dot_general(
            oh, x_blk, (((1,), (0,)), ((), ())), preferred_element_type=f32)
        ncnt = ncnt + lax.dot_general(
            oh, ones_n, (((1,), (0,)), ((), ())), preferred_element_type=f32)
        return nsum, ncnt

    nsum, ncnt = lax.fori_loop(
        0, NBN, nstep, (jnp.zeros((B, DX), f32), jnp.zeros((B, DE), f32)))

    btail = b_ref[pl.ds(NBN, 1), pl.ds(0, NTAIL)]
    oh_t = (jnp.broadcast_to(btail, (B, NTAIL))
            == lax.broadcasted_iota(jnp.int32, (B, NTAIL), 0)).astype(f32)
    x_t = x_ref[pl.ds(NBN * 128, NTAIL), :]
    nsum = nsum + lax.dot_general(
        oh_t, x_t, (((1,), (0,)), ((), ())), preferred_element_type=f32)
    ncnt = ncnt + lax.dot_general(
        oh_t, jnp.ones((NTAIL, DE), f32), (((1,), (0,)), ((), ())),
        preferred_element_type=f32)

    nmean = nsum / jnp.maximum(ncnt[:, 0:1], 1.0)

    es = es_ref[0] + es_ref[1]
    ec = ec_ref[0] + ec_ref[1]
    emean = es / jnp.maximum(ec, 1.0)

    z = (jnp.dot(u_ref[...], w1u_ref[...], preferred_element_type=f32)
         + jnp.dot(nmean, w1x_ref[...], preferred_element_type=f32)
         + jnp.dot(emean, w1e_ref[...], preferred_element_type=f32)
         + b1_ref[...])
    h = jnp.maximum(z, 0.0)
    out_ref[...] = jnp.dot(h, w2_ref[...], preferred_element_type=f32) \
        + b2_ref[...]


def _tc_fuse(x, bp, u, w1u, w1x, w1e, b1, w2, b2, esum, ecnt):
    return pl.pallas_call(
        _tc_fuse_kernel,
        out_shape=jax.ShapeDtypeStruct((B, H2), jnp.float32),
    )(x, bp, u, w1u, w1x, w1e, b1, w2, b2, esum, ecnt)


@jax.jit
def kernel(x, edge_index, edge_attr, u, batch, W1, b1, W2, b2):
    row = edge_index[0].astype(jnp.int32)
    batch32 = batch.astype(jnp.int32)

    esum, ecnt = _sc_edge(row, edge_attr, batch32)
    ecnt = ecnt.reshape(NC, B, 1)

    bp = jnp.pad(batch32, (0, NPAD - N), constant_values=-1).reshape(-1, 128)
    w1u = W1[:DU]
    w1x = W1[DU:DU + DX]
    w1e = W1[DU + DX:]
    return _tc_fuse(x, bp, u, w1u, w1x, w1e, b1.reshape(1, H1), W2,
                    b2.reshape(1, H2), esum, ecnt)


# speedup vs baseline: 1.4530x; 1.0675x over previous
import jax
import jax.numpy as jnp
from jax import lax
from jax.experimental import pallas as pl
from jax.experimental.pallas import tpu as pltpu
from jax.experimental.pallas import tpu_sc as plsc

N = 10000
E = 320000
B = 64
DX = 128
DE = 16
DU = 128
H1 = 512
H2 = 256

NC = 2
NS = 16
E_PER_W = E // (NC * NS)
KG = 2000
S_PER_W = E_PER_W // KG


def _sc_edge_kernel(row_hbm, attr_hbm, batch_hbm, out_sum, out_cnt,
                    batch_v, rows_v, attr_v, acc_v, acc2_v, cnt_v, tmp_v,
                    tmpc_v, sum_sh, cnt_sh):
    c = lax.axis_index("c")
    s = lax.axis_index("s")
    wid = s * NC + c

    def zrow(i, _):
        acc_v[i, :] = jnp.zeros((16,), jnp.float32)
        acc2_v[i, :] = jnp.zeros((16,), jnp.float32)
        return 0
    lax.fori_loop(0, B, zrow, 0)
    for i in range(B // 16):
        cnt_v[pl.ds(16 * i, 16)] = jnp.zeros((16,), jnp.float32)

    pltpu.sync_copy(batch_hbm, batch_v)
    ones16 = jnp.ones((16,), jnp.float32)
    iota16 = lax.broadcasted_iota(jnp.int32, (16,), 0)
    base = wid * E_PER_W

    def chunk(si, _):
        off = base + si * KG
        pltpu.sync_copy(row_hbm.at[pl.ds(off, KG)], rows_v)
        pltpu.sync_copy(attr_hbm.at[pl.ds(off // 8, KG // 8)], attr_v)

        def grp(i, _):
            r = rows_v[pl.ds(16 * i, 16)]
            seg = plsc.load_gather(batch_v, [r])
            plsc.addupdate_scatter(cnt_v, [seg], ones16)
            for kk in range(0, 16, 2):
                rowa = attr_v[2 * i + kk // 8, pl.ds((kk % 8) * 16, 16)]
                rowb = attr_v[2 * i + (kk + 1) // 8,
                              pl.ds(((kk + 1) % 8) * 16, 16)]
                plsc.addupdate(acc_v.at[seg[kk]], rowa)
                plsc.addupdate(acc2_v.at[seg[kk + 1]], rowb)
            return 0
        lax.fori_loop(0, KG // 16, grp, 0)
        return 0
    lax.fori_loop(0, S_PER_W, chunk, 0)

    def mrow(i, _):
        acc_v[i, :] = acc_v[i, :] + acc2_v[i, :]
        return 0
    lax.fori_loop(0, B, mrow, 0)

    pltpu.sync_copy(acc_v, sum_sh.at[s])
    pltpu.sync_copy(cnt_v, cnt_sh.at[s])
    plsc.subcore_barrier()

    @pl.when(s == 0)
    def _():
        def red(t, _):
            pltpu.sync_copy(sum_sh.at[t], tmp_v)
            def addrow(i, _):
                acc_v[i, :] = acc_v[i, :] + tmp_v[i, :]
                return 0
            lax.fori_loop(0, B, addrow, 0)
            pltpu.sync_copy(cnt_sh.at[t], tmpc_v)
            for i in range(B // 16):
                sl = pl.ds(16 * i, 16)
                cnt_v[sl] = cnt_v[sl] + tmpc_v[sl]
            return 0
        lax.fori_loop(1, NS, red, 0)
        pltpu.sync_copy(acc_v, out_sum.at[c])
        pltpu.sync_copy(cnt_v, out_cnt.at[c])


def _sc_edge(row, edge_attr, batch32):
    mesh = plsc.VectorSubcoreMesh(core_axis_name="c", subcore_axis_name="s")
    f32 = jnp.float32
    return pl.kernel(
        _sc_edge_kernel,
        out_type=(jax.ShapeDtypeStruct((NC, B, DE), f32),
                  jax.ShapeDtypeStruct((NC, B), f32)),
        mesh=mesh,
        compiler_params=pltpu.CompilerParams(
            needs_layout_passes=False, use_tc_tiling_on_sc=False),
        scratch_types=[
            pltpu.VMEM((N,), jnp.int32),
            pltpu.VMEM((KG,), jnp.int32),
            pltpu.VMEM((KG // 8, 128), f32),
            pltpu.VMEM((B, DE), f32),
            pltpu.VMEM((B, DE), f32),
            pltpu.VMEM((B,), f32),
            pltpu.VMEM((B, DE), f32),
            pltpu.VMEM((B,), f32),
            pltpu.VMEM_SHARED((NS, B, DE), f32),
            pltpu.VMEM_SHARED((NS, B), f32),
        ],
    )(row, edge_attr, batch32)


NBN = N // 128
NTAIL = N - NBN * 128
NPAD = (NBN + 2) * 128


def _tc_fuse_kernel(x_ref, b_ref, u_ref, w1u_ref, w1x_ref, w1e_ref, b1_ref,
                    w2_ref, b2_ref, es_ref, ec_ref, out_ref):
    f32 = jnp.float32
    bins = lax.broadcasted_iota(jnp.int32, (B, 128), 0)
    ones_n = jnp.ones((128, DE), f32)

    def nstep(j, carry):
        nsum, ncnt = carry
        brow = b_ref[pl.ds(j, 1), :]
        oh = (jnp.broadcast_to(brow, (B, 128)) == bins).astype(f32)
        x_blk = x_ref[pl.ds(j * 128, 128), :]
        nsum = nsum + lax.dot_general(
            oh, x_blk, (((1,), (0,)), ((), ())), preferred_element_type=f32)
        ncnt = ncnt + lax.dot_general(
            oh, ones_n, (((1,), (0,)), ((), ())), preferred_element_type=f32)
        return nsum, ncnt

    nsum, ncnt = lax.fori_loop(
        0, NBN, nstep, (jnp.zeros((B, DX), f32), jnp.zeros((B, DE), f32)))

    btail = b_ref[pl.ds(NBN, 1), pl.ds(0, NTAIL)]
    oh_t = (jnp.broadcast_to(btail, (B, NTAIL))
            == lax.broadcasted_iota(jnp.int32, (B, NTAIL), 0)).astype(f32)
    x_t = x_ref[pl.ds(NBN * 128, NTAIL), :]
    nsum = nsum + lax.dot_general(
        oh_t, x_t, (((1,), (0,)), ((), ())), preferred_element_type=f32)
    ncnt = ncnt + lax.dot_general(
        oh_t, jnp.ones((NTAIL, DE), f32), (((1,), (0,)), ((), ())),
        preferred_element_type=f32)

    nmean = nsum / jnp.maximum(ncnt[:, 0:1], 1.0)

    es = es_ref[0] + es_ref[1]
    ec = ec_ref[0] + ec_ref[1]
    emean = es / jnp.maximum(ec, 1.0)

    z = (jnp.dot(u_ref[...], w1u_ref[...], preferred_element_type=f32)
         + jnp.dot(nmean, w1x_ref[...], preferred_element_type=f32)
         + jnp.dot(emean, w1e_ref[...], preferred_element_type=f32)
         + b1_ref[...])
    h = jnp.maximum(z, 0.0)
    out_ref[...] = jnp.dot(h, w2_ref[...], preferred_element_type=f32) \
        + b2_ref[...]


def _tc_fuse(x, bp, u, w1u, w1x, w1e, b1, w2, b2, esum, ecnt):
    return pl.pallas_call(
        _tc_fuse_kernel,
        out_shape=jax.ShapeDtypeStruct((B, H2), jnp.float32),
    )(x, bp, u, w1u, w1x, w1e, b1, w2, b2, esum, ecnt)


@jax.jit
def kernel(x, edge_index, edge_attr, u, batch, W1, b1, W2, b2):
    row = edge_index[0].astype(jnp.int32)
    batch32 = batch.astype(jnp.int32)

    esum, ecnt = _sc_edge(row, edge_attr.reshape(E // 8, 128), batch32)
    ecnt = ecnt.reshape(NC, B, 1)

    bp = jnp.pad(batch32, (0, NPAD - N), constant_values=-1).reshape(-1, 128)
    w1u = W1[:DU]
    w1x = W1[DU:DU + DX]
    w1e = W1[DU + DX:]
    return _tc_fuse(x, bp, u, w1u, w1x, w1e, b1.reshape(1, H1), W2,
                    b2.reshape(1, H2), esum, ecnt)


# speedup vs baseline: 1.7126x; 1.1787x over previous
import jax
import jax.numpy as jnp
from jax import lax
from jax.experimental import pallas as pl
from jax.experimental.pallas import tpu as pltpu
from jax.experimental.pallas import tpu_sc as plsc

N = 10000
E = 320000
B = 64
DX = 128
DE = 16
DU = 128
H1 = 512
H2 = 256

NC = 2
NS = 16
E_PER_W = E // (NC * NS)
KG = 2000
S_PER_W = E_PER_W // KG


NCH = E // KG


def _sc_edge_kernel(row_hbm, attrt_hbm, batch_hbm, out_sum, out_cnt,
                    batch_v, rows_v, attrt_v, acct_v, cnt_v, tmpt_v,
                    tmpc_v, sumt_sh, cnt_sh):
    c = lax.axis_index("c")
    s = lax.axis_index("s")
    wid = s * NC + c
    z16 = jnp.zeros((16,), jnp.float32)

    for i in range(DE):
        for j in range(B // 16):
            acct_v[i, pl.ds(16 * j, 16)] = z16
    for j in range(B // 16):
        cnt_v[pl.ds(16 * j, 16)] = z16

    pltpu.sync_copy(batch_hbm, batch_v)
    ones16 = jnp.ones((16,), jnp.float32)
    base = wid * E_PER_W

    def chunk(si, _):
        off = base + si * KG
        pltpu.sync_copy(row_hbm.at[pl.ds(off, KG)], rows_v)
        pltpu.sync_copy(attrt_hbm.at[wid * S_PER_W + si], attrt_v)

        def grp(i, _):
            r = rows_v[pl.ds(16 * i, 16)]
            seg = plsc.load_gather(batch_v, [r])
            plsc.addupdate_scatter(cnt_v, [seg], ones16)
            for f in range(DE):
                col = attrt_v[f, pl.ds(16 * i, 16)]
                plsc.addupdate_scatter(acct_v.at[f], [seg], col)
            return 0
        lax.fori_loop(0, KG // 16, grp, 0)
        return 0
    lax.fori_loop(0, S_PER_W, chunk, 0)

    pltpu.sync_copy(acct_v, sumt_sh.at[s])
    pltpu.sync_copy(cnt_v, cnt_sh.at[s])
    plsc.subcore_barrier()

    @pl.when(s == 0)
    def _():
        def red(t, _):
            pltpu.sync_copy(sumt_sh.at[t], tmpt_v)
            def addrow(i, _):
                def addcol(j, _):
                    sl = pl.ds(16 * j, 16)
                    acct_v[i, sl] = acct_v[i, sl] + tmpt_v[i, sl]
                    return 0
                lax.fori_loop(0, B // 16, addcol, 0)
                return 0
            lax.fori_loop(0, DE, addrow, 0)
            pltpu.sync_copy(cnt_sh.at[t], tmpc_v)
            for j in range(B // 16):
                sl = pl.ds(16 * j, 16)
                cnt_v[sl] = cnt_v[sl] + tmpc_v[sl]
            return 0
        lax.fori_loop(1, NS, red, 0)
        pltpu.sync_copy(acct_v, out_sum.at[c])
        pltpu.sync_copy(cnt_v, out_cnt.at[c])


def _sc_edge(row, attrt, batch32):
    mesh = plsc.VectorSubcoreMesh(core_axis_name="c", subcore_axis_name="s")
    f32 = jnp.float32
    return pl.kernel(
        _sc_edge_kernel,
        out_type=(jax.ShapeDtypeStruct((NC, DE, B), f32),
                  jax.ShapeDtypeStruct((NC, B), f32)),
        mesh=mesh,
        compiler_params=pltpu.CompilerParams(
            needs_layout_passes=False, use_tc_tiling_on_sc=False),
        scratch_types=[
            pltpu.VMEM((N,), jnp.int32),
            pltpu.VMEM((KG,), jnp.int32),
            pltpu.VMEM((DE, KG), f32),
            pltpu.VMEM((DE, B), f32),
            pltpu.VMEM((B,), f32),
            pltpu.VMEM((DE, B), f32),
            pltpu.VMEM((B,), f32),
            pltpu.VMEM_SHARED((NS, DE, B), f32),
            pltpu.VMEM_SHARED((NS, B), f32),
        ],
    )(row, attrt, batch32)


NBN = N // 128
NTAIL = N - NBN * 128
NPAD = (NBN + 2) * 128


def _tc_fuse_kernel(x_ref, b_ref, u_ref, w1u_ref, w1x_ref, w1e_ref, b1_ref,
                    w2_ref, b2_ref, es_ref, ec_ref, out_ref):
    f32 = jnp.float32
    bins = lax.broadcasted_iota(jnp.int32, (B, 128), 0)
    ones_n = jnp.ones((128, DE), f32)

    def nstep(j, carry):
        nsum, ncnt = carry
        brow = b_ref[pl.ds(j, 1), :]
        oh = (jnp.broadcast_to(brow, (B, 128)) == bins).astype(f32)
        x_blk = x_ref[pl.ds(j * 128, 128), :]
        nsum = nsum + lax.dot_general(
            oh, x_blk, (((1,), (0,)), ((), ())), preferred_element_type=f32)
        ncnt = ncnt + lax.dot_general(
            oh, ones_n, (((1,), (0,)), ((), ())), preferred_element_type=f32)
        return nsum, ncnt

    nsum, ncnt = lax.fori_loop(
        0, NBN, nstep, (jnp.zeros((B, DX), f32), jnp.zeros((B, DE), f32)))

    btail = b_ref[pl.ds(NBN, 1), pl.ds(0, NTAIL)]
    oh_t = (jnp.broadcast_to(btail, (B, NTAIL))
            == lax.broadcasted_iota(jnp.int32, (B, NTAIL), 0)).astype(f32)
    x_t = x_ref[pl.ds(NBN * 128, NTAIL), :]
    nsum = nsum + lax.dot_general(
        oh_t, x_t, (((1,), (0,)), ((), ())), preferred_element_type=f32)
    ncnt = ncnt + lax.dot_general(
        oh_t, jnp.ones((NTAIL, DE), f32), (((1,), (0,)), ((), ())),
        preferred_element_type=f32)

    nmean = nsum / jnp.maximum(ncnt[:, 0:1], 1.0)

    es_t = es_ref[0] + es_ref[1]
    ec = ec_ref[0] + ec_ref[1]
    emean_t = es_t / jnp.maximum(ec, 1.0)

    z = (jnp.dot(u_ref[...], w1u_ref[...], preferred_element_type=f32)
         + jnp.dot(nmean, w1x_ref[...], preferred_element_type=f32)
         + lax.dot_general(emean_t, w1e_ref[...], (((0,), (0,)), ((), ())),
                           preferred_element_type=f32)
         + b1_ref[...])
    h = jnp.maximum(z, 0.0)
    out_ref[...] = jnp.dot(h, w2_ref[...], preferred_element_type=f32) \
        + b2_ref[...]


def _tc_fuse(x, bp, u, w1u, w1x, w1e, b1, w2, b2, esum, ecnt):
    return pl.pallas_call(
        _tc_fuse_kernel,
        out_shape=jax.ShapeDtypeStruct((B, H2), jnp.float32),
    )(x, bp, u, w1u, w1x, w1e, b1, w2, b2, esum, ecnt)


@jax.jit
def kernel(x, edge_index, edge_attr, u, batch, W1, b1, W2, b2):
    row = edge_index[0].astype(jnp.int32)
    batch32 = batch.astype(jnp.int32)

    attrt = edge_attr.reshape(NCH, KG, DE).transpose(0, 2, 1)
    esum, ecnt = _sc_edge(row, attrt, batch32)
    ecnt = ecnt.reshape(NC, 1, B)

    bp = jnp.pad(batch32, (0, NPAD - N), constant_values=-1).reshape(-1, 128)
    w1u = W1[:DU]
    w1x = W1[DU:DU + DX]
    w1e = W1[DU + DX:]
    return _tc_fuse(x, bp, u, w1u, w1x, w1e, b1.reshape(1, H1), W2,
                    b2.reshape(1, H2), esum, ecnt)


# speedup vs baseline: 1.8195x; 1.0625x over previous
import jax
import jax.numpy as jnp
from jax import lax
from jax.experimental import pallas as pl
from jax.experimental.pallas import tpu as pltpu
from jax.experimental.pallas import tpu_sc as plsc

N = 10000
E = 320000
B = 64
DX = 128
DE = 16
DU = 128
H1 = 512
H2 = 256

NC = 2
NS = 16
E_PER_W = E // (NC * NS)
KG = 2000
S_PER_W = E_PER_W // KG


NCH = E // KG


def _sc_edge_kernel(row_hbm, attrt_hbm, batch_hbm, out_sum, out_cnt,
                    batch_v, rows_v, attrt_v, acct_v, cnt_v, tmpt_v,
                    tmpc_v, sumt_sh, cnt_sh):
    c = lax.axis_index("c")
    s = lax.axis_index("s")
    wid = s * NC + c
    z16 = jnp.zeros((16,), jnp.float32)

    for i in range(DE):
        for j in range(B // 16):
            acct_v[i, pl.ds(16 * j, 16)] = z16
    for j in range(B // 16):
        cnt_v[pl.ds(16 * j, 16)] = z16

    pltpu.sync_copy(batch_hbm, batch_v)
    ones16 = jnp.ones((16,), jnp.float32)
    base = wid * E_PER_W

    def chunk(si, _):
        off = base + si * KG
        pltpu.sync_copy(row_hbm.at[pl.ds(off, KG)], rows_v)
        pltpu.sync_copy(attrt_hbm.at[wid * S_PER_W + si], attrt_v)

        def grp(i, _):
            r = rows_v[pl.ds(16 * i, 16)]
            seg = plsc.load_gather(batch_v, [r])
            plsc.addupdate_scatter(cnt_v, [seg], ones16)
            for f in range(DE):
                col = attrt_v[f, pl.ds(16 * i, 16)]
                plsc.addupdate_scatter(acct_v.at[f], [seg], col)
            return 0
        lax.fori_loop(0, KG // 16, grp, 0)
        return 0
    lax.fori_loop(0, S_PER_W, chunk, 0)

    pltpu.sync_copy(acct_v, sumt_sh.at[s])
    pltpu.sync_copy(cnt_v, cnt_sh.at[s])
    plsc.subcore_barrier()

    @pl.when(s == 0)
    def _():
        def red(t, _):
            pltpu.sync_copy(sumt_sh.at[t], tmpt_v)
            def addrow(i, _):
                def addcol(j, _):
                    sl = pl.ds(16 * j, 16)
                    acct_v[i, sl] = acct_v[i, sl] + tmpt_v[i, sl]
                    return 0
                lax.fori_loop(0, B // 16, addcol, 0)
                return 0
            lax.fori_loop(0, DE, addrow, 0)
            pltpu.sync_copy(cnt_sh.at[t], tmpc_v)
            for j in range(B // 16):
                sl = pl.ds(16 * j, 16)
                cnt_v[sl] = cnt_v[sl] + tmpc_v[sl]
            return 0
        lax.fori_loop(1, NS, red, 0)
        pltpu.sync_copy(acct_v, out_sum.at[c])
        pltpu.sync_copy(cnt_v, out_cnt.at[c])


def _sc_edge(row, attrt, batch32):
    mesh = plsc.VectorSubcoreMesh(core_axis_name="c", subcore_axis_name="s")
    f32 = jnp.float32
    return pl.kernel(
        _sc_edge_kernel,
        out_type=(jax.ShapeDtypeStruct((NC, DE, B), f32),
                  jax.ShapeDtypeStruct((NC, B), f32)),
        mesh=mesh,
        compiler_params=pltpu.CompilerParams(
            needs_layout_passes=False, use_tc_tiling_on_sc=False),
        scratch_types=[
            pltpu.VMEM((N,), jnp.int32),
            pltpu.VMEM((KG,), jnp.int32),
            pltpu.VMEM((DE, KG), f32),
            pltpu.VMEM((DE, B), f32),
            pltpu.VMEM((B,), f32),
            pltpu.VMEM((DE, B), f32),
            pltpu.VMEM((B,), f32),
            pltpu.VMEM_SHARED((NS, DE, B), f32),
            pltpu.VMEM_SHARED((NS, B), f32),
        ],
    )(row, attrt, batch32)


NBN = N // 128
NTAIL = N - NBN * 128
NPAD = (NBN + 2) * 128


NMAIN = NBN * 128


def _tc_fuse_kernel(x_ref, bm_ref, bt_ref, u_ref, w1_ref, b1_ref,
                    w2_ref, b2_ref, es_ref, ec_ref, out_ref):
    f32 = jnp.float32

    bins = lax.broadcasted_iota(jnp.int32, (B, NMAIN), 0)
    oh = (jnp.broadcast_to(bm_ref[...], (B, NMAIN)) == bins).astype(f32)
    nsum = lax.dot_general(oh, x_ref[pl.ds(0, NMAIN), :],
                           (((1,), (0,)), ((), ())),
                           preferred_element_type=f32)
    ncnt = jnp.sum(oh, axis=1, keepdims=True)

    bins_t = lax.broadcasted_iota(jnp.int32, (B, NTAIL), 0)
    oh_t = (jnp.broadcast_to(bt_ref[...], (B, NTAIL)) == bins_t).astype(f32)
    nsum = nsum + lax.dot_general(
        oh_t, x_ref[pl.ds(NMAIN, NTAIL), :], (((1,), (0,)), ((), ())),
        preferred_element_type=f32)
    ncnt = ncnt + jnp.sum(oh_t, axis=1, keepdims=True)

    nmean = nsum / jnp.maximum(ncnt, 1.0)

    es_t = es_ref[0] + es_ref[1]
    ec = ec_ref[0] + ec_ref[1]
    emean_t = es_t / jnp.maximum(ec, 1.0)

    w1u = w1_ref[pl.ds(0, DU), :]
    w1x = w1_ref[pl.ds(DU, DX), :]
    w1e = w1_ref[pl.ds(DU + DX, DE), :]
    z = (jnp.dot(u_ref[...], w1u, preferred_element_type=f32)
         + jnp.dot(nmean, w1x, preferred_element_type=f32)
         + lax.dot_general(emean_t, w1e, (((0,), (0,)), ((), ())),
                           preferred_element_type=f32)
         + b1_ref[...])
    h = jnp.maximum(z, 0.0)
    out_ref[...] = jnp.dot(h, w2_ref[...], preferred_element_type=f32) \
        + b2_ref[...]


def _tc_fuse(x, bm, bt, u, w1, b1, w2, b2, esum, ecnt):
    return pl.pallas_call(
        _tc_fuse_kernel,
        out_shape=jax.ShapeDtypeStruct((B, H2), jnp.float32),
    )(x, bm, bt, u, w1, b1, w2, b2, esum, ecnt)


@jax.jit
def kernel(x, edge_index, edge_attr, u, batch, W1, b1, W2, b2):
    row = edge_index[0].astype(jnp.int32)
    batch32 = batch.astype(jnp.int32)

    attrt = edge_attr.reshape(NCH, KG, DE).transpose(0, 2, 1)
    esum, ecnt = _sc_edge(row, attrt, batch32)
    ecnt = ecnt.reshape(NC, 1, B)

    bm = batch32[:NMAIN].reshape(1, NMAIN)
    bt = batch32[NMAIN:].reshape(1, NTAIL)
    return _tc_fuse(x, bm, bt, u, W1, b1.reshape(1, H1), W2,
                    b2.reshape(1, H2), esum, ecnt)


# speedup vs baseline: 2.1649x; 1.1898x over previous
import jax
import jax.numpy as jnp
from jax import lax
from jax.experimental import pallas as pl
from jax.experimental.pallas import tpu as pltpu
from jax.experimental.pallas import tpu_sc as plsc

N = 10000
E = 320000
B = 64
DX = 128
DE = 16
DU = 128
H1 = 512
H2 = 256

NC = 2
NS = 16
E_PER_W = E // (NC * NS)
KG = 2000
S_PER_W = E_PER_W // KG


NCH = E // KG


def _sc_edge_kernel(row_hbm, attrt_hbm, batch_hbm, out_sum, out_cnt,
                    batch_v, rows_v, attrt_v, acct_v, acct2_v, cnt_v,
                    tmpt_v, tmpc_v, sumt_sh, cnt_sh, sem0, sem1):
    c = lax.axis_index("c")
    s = lax.axis_index("s")
    wid = s * NC + c
    z16 = jnp.zeros((16,), jnp.float32)

    for i in range(DE):
        for j in range(B // 16):
            acct_v[i, pl.ds(16 * j, 16)] = z16
            acct2_v[i, pl.ds(16 * j, 16)] = z16
    for j in range(B // 16):
        cnt_v[pl.ds(16 * j, 16)] = z16

    ones16 = jnp.ones((16,), jnp.float32)
    base = wid * E_PER_W
    sems = (sem0, sem1)

    def start(si, p):
        off = base + si * KG
        d1 = pltpu.async_copy(row_hbm.at[pl.ds(off, KG)], rows_v.at[p],
                              sems[p])
        d2 = pltpu.async_copy(attrt_hbm.at[wid * S_PER_W + si],
                              attrt_v.at[p], sems[p])
        return (d1, d2)

    d = start(0, 0)
    pltpu.sync_copy(batch_hbm, batch_v)

    for si in range(S_PER_W):
        p = si % 2
        d[0].wait()
        d[1].wait()
        if si + 1 < S_PER_W:
            d = start(si + 1, 1 - p)

        def grp(i, _):
            r0 = rows_v[p, pl.ds(32 * i, 16)]
            r1 = rows_v[p, pl.ds(32 * i + 16, 16)]
            s0 = plsc.load_gather(batch_v, [r0])
            s1 = plsc.load_gather(batch_v, [r1])
            plsc.addupdate_scatter(cnt_v, [s0], ones16)
            plsc.addupdate_scatter(cnt_v, [s1], ones16)
            for f in range(DE):
                col0 = attrt_v[p, f, pl.ds(32 * i, 16)]
                col1 = attrt_v[p, f, pl.ds(32 * i + 16, 16)]
                plsc.addupdate_scatter(acct_v.at[f], [s0], col0)
                plsc.addupdate_scatter(acct2_v.at[f], [s1], col1)
            return 0
        lax.fori_loop(0, KG // 32, grp, 0)

    for i in range(DE):
        for j in range(B // 16):
            sl = pl.ds(16 * j, 16)
            acct_v[i, sl] = acct_v[i, sl] + acct2_v[i, sl]

    pltpu.sync_copy(acct_v, sumt_sh.at[s])
    pltpu.sync_copy(cnt_v, cnt_sh.at[s])
    plsc.subcore_barrier()

    @pl.when(s == 0)
    def _():
        def red(t, _):
            pltpu.sync_copy(sumt_sh.at[t], tmpt_v)
            def addrow(i, _):
                def addcol(j, _):
                    sl = pl.ds(16 * j, 16)
                    acct_v[i, sl] = acct_v[i, sl] + tmpt_v[i, sl]
                    return 0
                lax.fori_loop(0, B // 16, addcol, 0)
                return 0
            lax.fori_loop(0, DE, addrow, 0)
            pltpu.sync_copy(cnt_sh.at[t], tmpc_v)
            for j in range(B // 16):
                sl = pl.ds(16 * j, 16)
                cnt_v[sl] = cnt_v[sl] + tmpc_v[sl]
            return 0
        lax.fori_loop(1, NS, red, 0)
        pltpu.sync_copy(acct_v, out_sum.at[c])
        pltpu.sync_copy(cnt_v, out_cnt.at[c])


def _sc_edge(row, attrt, batch32):
    mesh = plsc.VectorSubcoreMesh(core_axis_name="c", subcore_axis_name="s")
    f32 = jnp.float32
    return pl.kernel(
        _sc_edge_kernel,
        out_type=(jax.ShapeDtypeStruct((NC, DE, B), f32),
                  jax.ShapeDtypeStruct((NC, B), f32)),
        mesh=mesh,
        compiler_params=pltpu.CompilerParams(
            needs_layout_passes=False, use_tc_tiling_on_sc=False),
        scratch_types=[
            pltpu.VMEM((N,), jnp.int32),
            pltpu.VMEM((2, KG), jnp.int32),
            pltpu.VMEM((2, DE, KG), f32),
            pltpu.VMEM((DE, B), f32),
            pltpu.VMEM((DE, B), f32),
            pltpu.VMEM((B,), f32),
            pltpu.VMEM((DE, B), f32),
            pltpu.VMEM((B,), f32),
            pltpu.VMEM_SHARED((NS, DE, B), f32),
            pltpu.VMEM_SHARED((NS, B), f32),
            pltpu.SemaphoreType.DMA,
            pltpu.SemaphoreType.DMA,
        ],
    )(row, attrt, batch32)


NBN = N // 128
NTAIL = N - NBN * 128
NPAD = (NBN + 2) * 128


NMAIN = NBN * 128


def _tc_fuse_kernel(x_ref, bm_ref, bt_ref, u_ref, w1_ref, b1_ref,
                    w2_ref, b2_ref, es_ref, ec_ref, out_ref):
    f32 = jnp.float32

    bins = lax.broadcasted_iota(jnp.int32, (B, NMAIN), 0)
    oh = (jnp.broadcast_to(bm_ref[...], (B, NMAIN)) == bins).astype(f32)
    nsum = lax.dot_general(oh, x_ref[pl.ds(0, NMAIN), :],
                           (((1,), (0,)), ((), ())),
                           preferred_element_type=f32)
    ncnt = jnp.sum(oh, axis=1, keepdims=True)

    bins_t = lax.broadcasted_iota(jnp.int32, (B, NTAIL), 0)
    oh_t = (jnp.broadcast_to(bt_ref[...], (B, NTAIL)) == bins_t).astype(f32)
    nsum = nsum + lax.dot_general(
        oh_t, x_ref[pl.ds(NMAIN, NTAIL), :], (((1,), (0,)), ((), ())),
        preferred_element_type=f32)
    ncnt = ncnt + jnp.sum(oh_t, axis=1, keepdims=True)

    nmean = nsum / jnp.maximum(ncnt, 1.0)

    es_t = es_ref[0] + es_ref[1]
    ec = ec_ref[0] + ec_ref[1]
    emean_t = es_t / jnp.maximum(ec, 1.0)

    w1u = w1_ref[pl.ds(0, DU), :]
    w1x = w1_ref[pl.ds(DU, DX), :]
    w1e = w1_ref[pl.ds(DU + DX, DE), :]
    z = (jnp.dot(u_ref[...], w1u, preferred_element_type=f32)
         + jnp.dot(nmean, w1x, preferred_element_type=f32)
         + lax.dot_general(emean_t, w1e, (((0,), (0,)), ((), ())),
                           preferred_element_type=f32)
         + b1_ref[...])
    h = jnp.maximum(z, 0.0)
    out_ref[...] = jnp.dot(h, w2_ref[...], preferred_element_type=f32) \
        + b2_ref[...]


def _tc_fuse(x, bm, bt, u, w1, b1, w2, b2, esum, ecnt):
    return pl.pallas_call(
        _tc_fuse_kernel,
        out_shape=jax.ShapeDtypeStruct((B, H2), jnp.float32),
    )(x, bm, bt, u, w1, b1, w2, b2, esum, ecnt)


@jax.jit
def kernel(x, edge_index, edge_attr, u, batch, W1, b1, W2, b2):
    row = edge_index[0].astype(jnp.int32)
    batch32 = batch.astype(jnp.int32)

    attrt = edge_attr.reshape(NCH, KG, DE).transpose(0, 2, 1)
    esum, ecnt = _sc_edge(row, attrt, batch32)
    ecnt = ecnt.reshape(NC, 1, B)

    bm = batch32[:NMAIN].reshape(1, NMAIN)
    bt = batch32[NMAIN:].reshape(1, NTAIL)
    return _tc_fuse(x, bm, bt, u, W1, b1.reshape(1, H1), W2,
                    b2.reshape(1, H2), esum, ecnt)
